# baseline (speedup 1.0000x reference)
import functools

import jax
import jax.numpy as jnp
from jax import lax
from jax.experimental import pallas as pl
from jax.experimental.pallas import tpu as pltpu
from jax.experimental.pallas import tpu_sc as plsc

N = 10000
E = 320000
D = 128
NC = 2
NS = 16
NW = NC * NS
K = 128
N_PAD = 10240
PAD_T = N_PAD // NS
E_PAD = 323584
PAD_IDX = N + 1
CH_A = E_PAD // NS // K
CH_C = E_PAD // NW // K

_mesh = plsc.VectorSubcoreMesh(
    core_axis_name="c", subcore_axis_name="s", num_cores=NC, num_subcores=NS)


@functools.partial(
    pl.kernel,
    out_type=[jax.ShapeDtypeStruct((N_PAD,), jnp.float32),
              jax.ShapeDtypeStruct((N_PAD,), jnp.float32)],
    mesh=_mesh,
    scratch_types=[
        pltpu.VMEM_SHARED((N_PAD,), jnp.float32),
        pltpu.VMEM((PAD_T,), jnp.float32),
        pltpu.VMEM((K,), jnp.float32),
        pltpu.VMEM((2, CH_A, K), jnp.int32),
        pltpu.SemaphoreType.DMA,
    ],
)
def _sc_degrees(ei, deg_out, deg_in, hist, zb, ones, idxs, sem):
    cid = lax.axis_index("c")
    sid = lax.axis_index("s")

    def _z(i, _):
        zb[pl.ds(i * 16, 16)] = jnp.zeros((16,), jnp.float32)
        return 0
    lax.fori_loop(0, PAD_T // 16, _z, 0)

    def _o(i, _):
        ones[pl.ds(i * 16, 16)] = jnp.ones((16,), jnp.float32)
        return 0
    lax.fori_loop(0, K // 16, _o, 0)

    pltpu.sync_copy(zb, hist.at[pl.ds(sid * PAD_T, PAD_T)])
    pltpu.sync_copy(ei.at[:, sid], idxs)
    plsc.subcore_barrier()

    def _body(c, _):
        pltpu.async_copy(ones, hist.at[idxs.at[cid, c]], sem, add=True)
        return 0
    lax.fori_loop(0, CH_A, _body, 0)

    def _drain(c, _):
        pltpu.make_async_copy(ones, hist.at[idxs.at[cid, 0]], sem).wait()
        return 0
    lax.fori_loop(0, CH_A, _drain, 0)

    plsc.subcore_barrier()

    @pl.when(cid == 0)
    def _():
        pltpu.sync_copy(hist.at[pl.ds(sid * PAD_T, PAD_T)],
                        deg_out.at[pl.ds(sid * PAD_T, PAD_T)])

    @pl.when(cid == 1)
    def _():
        pltpu.sync_copy(hist.at[pl.ds(sid * PAD_T, PAD_T)],
                        deg_in.at[pl.ds(sid * PAD_T, PAD_T)])


RB = 1000
RP = N_PAD // 10


def _l1_body(f_ref, dop_ref, dip_ref, w1_ref, x1_ref, invo_ref, invi_ref):
    io = lax.rsqrt(jnp.maximum(dop_ref[...], 1.0))
    x1_ref[...] = jnp.dot(f_ref[...] * io, w1_ref[...],
                          preferred_element_type=jnp.float32)
    invo_ref[...] = io
    invi_ref[...] = lax.rsqrt(jnp.maximum(dip_ref[...], 1.0))


_tc_layer1 = pl.pallas_call(
    _l1_body,
    grid=(N_PAD // RP,),
    in_specs=[
        pl.BlockSpec((RP, D), lambda i: (i, 0)),
        pl.BlockSpec((RP, 1), lambda i: (i, 0)),
        pl.BlockSpec((RP, 1), lambda i: (i, 0)),
        pl.BlockSpec((D, D), lambda i: (0, 0)),
    ],
    out_specs=[
        pl.BlockSpec((RP, D), lambda i: (i, 0)),
        pl.BlockSpec((RP, 1), lambda i: (i, 0)),
        pl.BlockSpec((RP, 1), lambda i: (i, 0)),
    ],
    out_shape=[
        jax.ShapeDtypeStruct((N_PAD, D), jnp.float32),
        jax.ShapeDtypeStruct((N_PAD, 1), jnp.float32),
        jax.ShapeDtypeStruct((N_PAD, 1), jnp.float32),
    ],
)


@functools.partial(
    pl.kernel,
    out_type=[jax.ShapeDtypeStruct((NC, N_PAD, D), jnp.float32),
              jax.ShapeDtypeStruct((N_PAD,), jnp.float32),
              jax.ShapeDtypeStruct((N_PAD,), jnp.float32)],
    mesh=_mesh,
    scratch_types=[
        pltpu.VMEM_SHARED((N_PAD, D), jnp.float32),
        pltpu.VMEM_SHARED((N_PAD,), jnp.float32),
        pltpu.VMEM_SHARED((N_PAD,), jnp.float32),
        pltpu.VMEM((32, D), jnp.float32),
        pltpu.VMEM((PAD_T,), jnp.float32),
        pltpu.VMEM((2, 1, K), jnp.int32),
        pltpu.VMEM((2, 1, K), jnp.int32),
        pltpu.VMEM((2, 1, K), jnp.int32),
        pltpu.VMEM((K, D), jnp.float32),
        pltpu.VMEM((K, D), jnp.float32),
        pltpu.VMEM((K,), jnp.float32),
        pltpu.VMEM((K,), jnp.float32),
        pltpu.SemaphoreType.DMA,
        pltpu.SemaphoreType.DMA,
        pltpu.SemaphoreType.DMA,
        pltpu.SemaphoreType.DMA,
        pltpu.SemaphoreType.DMA,
        pltpu.SemaphoreType.DMA,
        pltpu.SemaphoreType.DMA,
        pltpu.SemaphoreType.DMA,
        pltpu.SemaphoreType.DMA,
        pltpu.SemaphoreType.DMA,
        pltpu.SemaphoreType.DMA,
    ],
)
def _sc_edge_pass(ei, x1, invin, aggp, sp0, sp1,
                  agg_sh, s_sh, inv_sh, zb2, zb1, i0, i1, i2,
                  r0, r1, v0, v1, si0, si1, si2, sr0, sr1, sv0, sv1,
                  ssr0, ssr1, ssv0, ssv1):
    cid = lax.axis_index("c")
    sid = lax.axis_index("s")

    def _zr(r, _):
        def _zc(c, _):
            zb2[r, pl.ds(c * 16, 16)] = jnp.zeros((16,), jnp.float32)
            return 0
        lax.fori_loop(0, D // 16, _zc, 0)
        return 0
    lax.fori_loop(0, 32, _zr, 0)

    def _z1(i, _):
        zb1[pl.ds(i * 16, 16)] = jnp.zeros((16,), jnp.float32)
        return 0
    lax.fori_loop(0, PAD_T // 16, _z1, 0)

    for j in range(PAD_T // 32):
        pltpu.sync_copy(zb2, agg_sh.at[pl.ds(sid * PAD_T + j * 32, 32), :])
    pltpu.sync_copy(zb1, s_sh.at[pl.ds(sid * PAD_T, PAD_T)])
    pltpu.sync_copy(invin.at[pl.ds(sid * PAD_T, PAD_T)],
                    inv_sh.at[pl.ds(sid * PAD_T, PAD_T)])

    idxs = (i0, i1, i2)
    rows = (r0, r1)
    vals = (v0, v1)
    isem = (si0, si1, si2)
    rsem = (sr0, sr1)
    vsem = (sv0, sv1)
    srsem = (ssr0, ssr1)
    svsem = (ssv0, ssv1)
    crow0 = (cid * NS + sid) * CH_C

    def load_idx(c, j):
        pltpu.async_copy(ei.at[:, crow0 + c], idxs[j], isem[j])

    def wait_idx(j):
        pltpu.make_async_copy(ei.at[:, 0], idxs[j], isem[j]).wait()

    def gath(j, b):
        pltpu.async_copy(x1.at[idxs[j].at[0, 0]], rows[b], rsem[b])
        pltpu.async_copy(inv_sh.at[idxs[j].at[1, 0]], vals[b], vsem[b])

    def drain_gath(j, b):
        pltpu.make_async_copy(x1.at[idxs[j].at[0, 0]], rows[b], rsem[b]).wait()
        pltpu.make_async_copy(inv_sh.at[idxs[j].at[1, 0]], vals[b],
                              vsem[b]).wait()

    def scat(j, b):
        pltpu.async_copy(rows[b], agg_sh.at[idxs[j].at[1, 0]], srsem[b],
                         add=True)
        pltpu.async_copy(vals[b], s_sh.at[idxs[j].at[0, 0]], svsem[b],
                         add=True)

    def drain_scat(b):
        pltpu.make_async_copy(rows[b], agg_sh.at[i0.at[1, 0]],
                              srsem[b]).wait()
        pltpu.make_async_copy(vals[b], s_sh.at[i0.at[0, 0]],
                              svsem[b]).wait()

    load_idx(0, 0)
    load_idx(1, 1)
    wait_idx(0)
    plsc.subcore_barrier()
    gath(0, 0)

    def stage(c, j, b, first=False, load=True, gather=True):
        if not first:
            drain_scat(b ^ 1)
        if load:
            load_idx(c + 2, (j + 2) % 3)
        if gather:
            wait_idx((j + 1) % 3)
            gath((j + 1) % 3, b ^ 1)
        drain_gath(j, b)
        scat(j, b)

    stage(0, 0, 0, first=True)

    def _body(m, _):
        c = 6 * m + 1
        for k in range(6):
            stage(c + k, (1 + k) % 3, (1 + k) % 2)
        return 0
    lax.fori_loop(0, (CH_C - 7) // 6, _body, 0)

    stage(CH_C - 6, 1, 1)
    stage(CH_C - 5, 2, 0)
    stage(CH_C - 4, 0, 1)
    stage(CH_C - 3, 1, 0)
    stage(CH_C - 2, 2, 1, load=False)
    stage(CH_C - 1, 0, 0, load=False, gather=False)
    drain_scat(0)

    plsc.subcore_barrier()
    pltpu.sync_copy(agg_sh.at[pl.ds(sid * PAD_T, PAD_T), :],
                    aggp.at[cid, pl.ds(sid * PAD_T, PAD_T), :])

    @pl.when(cid == 0)
    def _():
        pltpu.sync_copy(s_sh.at[pl.ds(sid * PAD_T, PAD_T)],
                        sp0.at[pl.ds(sid * PAD_T, PAD_T)])

    @pl.when(cid == 1)
    def _():
        pltpu.sync_copy(s_sh.at[pl.ds(sid * PAD_T, PAD_T)],
                        sp1.at[pl.ds(sid * PAD_T, PAD_T)])


def _fin_body(aggp_ref, sp0_ref, sp1_ref, invo_ref, invi_ref, b1_ref, w2_ref,
              b2_ref, out_ref, acc):
    i = pl.program_id(0)
    agg = aggp_ref[0] + aggp_ref[1]
    h = jnp.maximum(agg * invi_ref[...] + b1_ref[...], 0.0)
    w = invo_ref[...] * (sp0_ref[...] + sp1_ref[...])
    partial = jnp.sum(w * h, axis=0, keepdims=True)

    @pl.when(i == 0)
    def _():
        acc[...] = jnp.zeros_like(acc)

    acc[...] += partial

    @pl.when(i == N // RB - 1)
    def _():
        out_ref[...] = (jnp.dot(acc[...], w2_ref[...],
                                preferred_element_type=jnp.float32)
                        * (1.0 / N) + b2_ref[...])


_tc_finish = pl.pallas_call(
    _fin_body,
    grid=(N // RB,),
    in_specs=[
        pl.BlockSpec((NC, RB, D), lambda i: (0, i, 0)),
        pl.BlockSpec((RB, 1), lambda i: (i, 0)),
        pl.BlockSpec((RB, 1), lambda i: (i, 0)),
        pl.BlockSpec((RB, 1), lambda i: (i, 0)),
        pl.BlockSpec((RB, 1), lambda i: (i, 0)),
        pl.BlockSpec((1, D), lambda i: (0, 0)),
        pl.BlockSpec((D, D), lambda i: (0, 0)),
        pl.BlockSpec((1, D), lambda i: (0, 0)),
    ],
    out_specs=pl.BlockSpec((1, D), lambda i: (0, 0)),
    out_shape=jax.ShapeDtypeStruct((1, D), jnp.float32),
    scratch_shapes=[pltpu.VMEM((1, D), jnp.float32)],
)


def kernel(features, edge_index, W1, b1, W2, b2):
    ei = jnp.concatenate(
        [edge_index.astype(jnp.int32),
         jnp.full((2, E_PAD - E), PAD_IDX, jnp.int32)], axis=1)
    dego, degi = _sc_degrees(ei.reshape(2, NS, CH_A, K))
    x1, inv_out, inv_in = _tc_layer1(
        features, dego.reshape(N_PAD, 1), degi.reshape(N_PAD, 1), W1)
    aggp, sp0, sp1 = _sc_edge_pass(
        ei.reshape(2, E_PAD // K, 1, K), x1, inv_in.reshape(N_PAD))
    return _tc_finish(aggp, sp0.reshape(N_PAD, 1), sp1.reshape(N_PAD, 1),
                      inv_out, inv_in, b1.reshape(1, D), W2, b2.reshape(1, D))


# speedup vs baseline: 1.9837x; 1.9837x over previous
import functools

import jax
import jax.numpy as jnp
from jax import lax
from jax.experimental import pallas as pl
from jax.experimental.pallas import tpu as pltpu
from jax.experimental.pallas import tpu_sc as plsc

N = 10000
E = 320000
D = 128
NC = 2
NS = 16
NW = NC * NS
K = 128
N_PAD = 10240
PAD_T = N_PAD // NS
E_PAD = 323584
CH_A = E_PAD // NS // K
CH_C = E_PAD // NW // K

_mesh = plsc.VectorSubcoreMesh(
    core_axis_name="c", subcore_axis_name="s", num_cores=NC, num_subcores=NS)


@functools.partial(
    pl.kernel,
    out_type=[jax.ShapeDtypeStruct((N_PAD,), jnp.float32),
              jax.ShapeDtypeStruct((N_PAD,), jnp.float32)],
    mesh=_mesh,
    scratch_types=[
        pltpu.VMEM_SHARED((N_PAD,), jnp.float32),
        pltpu.VMEM((PAD_T,), jnp.float32),
        pltpu.VMEM((K,), jnp.float32),
        pltpu.VMEM((2, CH_A, K), jnp.int32),
        pltpu.SemaphoreType.DMA,
    ],
)
def _sc_degrees(ei, deg_out, deg_in, hist, zb, ones, idxs, sem):
    cid = lax.axis_index("c")
    sid = lax.axis_index("s")

    def _z(i, _):
        zb[pl.ds(i * 16, 16)] = jnp.zeros((16,), jnp.float32)
        return 0
    lax.fori_loop(0, PAD_T // 16, _z, 0)

    def _o(i, _):
        ones[pl.ds(i * 16, 16)] = jnp.ones((16,), jnp.float32)
        return 0
    lax.fori_loop(0, K // 16, _o, 0)

    pltpu.sync_copy(zb, hist.at[pl.ds(sid * PAD_T, PAD_T)])
    pltpu.sync_copy(ei.at[:, sid], idxs)
    plsc.subcore_barrier()

    def _body(c, _):
        pltpu.async_copy(ones, hist.at[idxs.at[cid, c]], sem, add=True)
        return 0
    lax.fori_loop(0, CH_A, _body, 0)

    def _drain(c, _):
        pltpu.make_async_copy(ones, hist.at[idxs.at[cid, 0]], sem).wait()
        return 0
    lax.fori_loop(0, CH_A, _drain, 0)

    plsc.subcore_barrier()

    @pl.when(cid == 0)
    def _():
        pltpu.sync_copy(hist.at[pl.ds(sid * PAD_T, PAD_T)],
                        deg_out.at[pl.ds(sid * PAD_T, PAD_T)])

    @pl.when(cid == 1)
    def _():
        pltpu.sync_copy(hist.at[pl.ds(sid * PAD_T, PAD_T)],
                        deg_in.at[pl.ds(sid * PAD_T, PAD_T)])


RB = 1000
RP = N_PAD // 10


def _l1_body(f_ref, dop_ref, dip_ref, w1_ref, x1_ref, invo_ref, invi_ref):
    io = lax.rsqrt(jnp.maximum(dop_ref[...], 1.0))
    x1_ref[...] = jnp.dot(f_ref[...] * io, w1_ref[...],
                          preferred_element_type=jnp.float32)
    invo_ref[...] = io
    invi_ref[...] = lax.rsqrt(jnp.maximum(dip_ref[...], 1.0))


_tc_layer1 = pl.pallas_call(
    _l1_body,
    grid=(N_PAD // RP,),
    in_specs=[
        pl.BlockSpec((RP, D), lambda i: (i, 0)),
        pl.BlockSpec((RP, 1), lambda i: (i, 0)),
        pl.BlockSpec((RP, 1), lambda i: (i, 0)),
        pl.BlockSpec((D, D), lambda i: (0, 0)),
    ],
    out_specs=[
        pl.BlockSpec((RP, D), lambda i: (i, 0)),
        pl.BlockSpec((RP, 1), lambda i: (i, 0)),
        pl.BlockSpec((RP, 1), lambda i: (i, 0)),
    ],
    out_shape=[
        jax.ShapeDtypeStruct((N_PAD, D), jnp.float32),
        jax.ShapeDtypeStruct((N_PAD, 1), jnp.float32),
        jax.ShapeDtypeStruct((N_PAD, 1), jnp.float32),
    ],
)


@functools.partial(
    pl.kernel,
    out_type=[jax.ShapeDtypeStruct((NC, N_PAD, D), jnp.float32),
              jax.ShapeDtypeStruct((N_PAD,), jnp.float32),
              jax.ShapeDtypeStruct((N_PAD,), jnp.float32)],
    mesh=_mesh,
    scratch_types=[
        pltpu.VMEM_SHARED((N_PAD, D), jnp.float32),
        pltpu.VMEM_SHARED((N_PAD,), jnp.float32),
        pltpu.VMEM_SHARED((N_PAD,), jnp.float32),
        pltpu.VMEM((32, D), jnp.float32),
        pltpu.VMEM((PAD_T,), jnp.float32),
        pltpu.VMEM((2, 1, K), jnp.int32),
        pltpu.VMEM((2, 1, K), jnp.int32),
        pltpu.VMEM((2, 1, K), jnp.int32),
        pltpu.VMEM((K, D), jnp.float32),
        pltpu.VMEM((K, D), jnp.float32),
        pltpu.VMEM((K,), jnp.float32),
        pltpu.VMEM((K,), jnp.float32),
        pltpu.SemaphoreType.DMA,
        pltpu.SemaphoreType.DMA,
        pltpu.SemaphoreType.DMA,
        pltpu.SemaphoreType.DMA,
        pltpu.SemaphoreType.DMA,
        pltpu.SemaphoreType.DMA,
        pltpu.SemaphoreType.DMA,
        pltpu.SemaphoreType.DMA,
        pltpu.SemaphoreType.DMA,
        pltpu.SemaphoreType.DMA,
        pltpu.SemaphoreType.DMA,
    ],
)
def _sc_edge_pass(ei, x1, invin, aggp, sp0, sp1,
                  agg_sh, s_sh, inv_sh, zb2, zb1, i0, i1, i2,
                  r0, r1, v0, v1, si0, si1, si2, sr0, sr1, sv0, sv1,
                  ssr0, ssr1, ssv0, ssv1):
    cid = lax.axis_index("c")
    sid = lax.axis_index("s")

    def _zr(r, _):
        def _zc(c, _):
            zb2[r, pl.ds(c * 16, 16)] = jnp.zeros((16,), jnp.float32)
            return 0
        lax.fori_loop(0, D // 16, _zc, 0)
        return 0
    lax.fori_loop(0, 32, _zr, 0)

    def _z1(i, _):
        zb1[pl.ds(i * 16, 16)] = jnp.zeros((16,), jnp.float32)
        return 0
    lax.fori_loop(0, PAD_T // 16, _z1, 0)

    for j in range(PAD_T // 32):
        pltpu.sync_copy(zb2, agg_sh.at[pl.ds(sid * PAD_T + j * 32, 32), :])
    pltpu.sync_copy(zb1, s_sh.at[pl.ds(sid * PAD_T, PAD_T)])
    pltpu.sync_copy(invin.at[pl.ds(sid * PAD_T, PAD_T)],
                    inv_sh.at[pl.ds(sid * PAD_T, PAD_T)])

    idxs = (i0, i1, i2)
    rows = (r0, r1)
    vals = (v0, v1)
    isem = (si0, si1, si2)
    rsem = (sr0, sr1)
    vsem = (sv0, sv1)
    srsem = (ssr0, ssr1)
    svsem = (ssv0, ssv1)
    crow0 = (cid * NS + sid) * CH_C

    def load_idx(c, j):
        pltpu.async_copy(ei.at[:, crow0 + c], idxs[j], isem[j])

    def wait_idx(j):
        pltpu.make_async_copy(ei.at[:, 0], idxs[j], isem[j]).wait()

    def gath(j, b):
        pltpu.async_copy(x1.at[idxs[j].at[0, 0]], rows[b], rsem[b])
        pltpu.async_copy(inv_sh.at[idxs[j].at[1, 0]], vals[b], vsem[b])

    def drain_gath(j, b):
        pltpu.make_async_copy(x1.at[idxs[j].at[0, 0]], rows[b], rsem[b]).wait()
        pltpu.make_async_copy(inv_sh.at[idxs[j].at[1, 0]], vals[b],
                              vsem[b]).wait()

    def scat(j, b):
        pltpu.async_copy(rows[b], agg_sh.at[idxs[j].at[1, 0]], srsem[b],
                         add=True)
        pltpu.async_copy(vals[b], s_sh.at[idxs[j].at[0, 0]], svsem[b],
                         add=True)

    def drain_scat(b):
        pltpu.make_async_copy(rows[b], agg_sh.at[i0.at[1, 0]],
                              srsem[b]).wait()
        pltpu.make_async_copy(vals[b], s_sh.at[i0.at[0, 0]],
                              svsem[b]).wait()

    load_idx(0, 0)
    load_idx(1, 1)
    wait_idx(0)
    plsc.subcore_barrier()
    gath(0, 0)

    def stage(c, j, b, first=False, load=True, gather=True):
        if not first:
            drain_scat(b ^ 1)
        if load:
            load_idx(c + 2, (j + 2) % 3)
        if gather:
            wait_idx((j + 1) % 3)
            gath((j + 1) % 3, b ^ 1)
        drain_gath(j, b)
        scat(j, b)

    stage(0, 0, 0, first=True)

    def _body(m, _):
        c = 6 * m + 1
        for k in range(6):
            stage(c + k, (1 + k) % 3, (1 + k) % 2)
        return 0
    lax.fori_loop(0, (CH_C - 7) // 6, _body, 0)

    stage(CH_C - 6, 1, 1)
    stage(CH_C - 5, 2, 0)
    stage(CH_C - 4, 0, 1)
    stage(CH_C - 3, 1, 0)
    stage(CH_C - 2, 2, 1, load=False)
    stage(CH_C - 1, 0, 0, load=False, gather=False)
    drain_scat(0)

    plsc.subcore_barrier()
    pltpu.sync_copy(agg_sh.at[pl.ds(sid * PAD_T, PAD_T), :],
                    aggp.at[cid, pl.ds(sid * PAD_T, PAD_T), :])

    @pl.when(cid == 0)
    def _():
        pltpu.sync_copy(s_sh.at[pl.ds(sid * PAD_T, PAD_T)],
                        sp0.at[pl.ds(sid * PAD_T, PAD_T)])

    @pl.when(cid == 1)
    def _():
        pltpu.sync_copy(s_sh.at[pl.ds(sid * PAD_T, PAD_T)],
                        sp1.at[pl.ds(sid * PAD_T, PAD_T)])


def _fin_body(aggp_ref, sp0_ref, sp1_ref, invo_ref, invi_ref, b1_ref, w2_ref,
              b2_ref, out_ref, acc):
    i = pl.program_id(0)
    agg = aggp_ref[0] + aggp_ref[1]
    h = jnp.maximum(agg * invi_ref[...] + b1_ref[...], 0.0)
    w = invo_ref[...] * (sp0_ref[...] + sp1_ref[...])
    partial = jnp.sum(w * h, axis=0, keepdims=True)

    @pl.when(i == 0)
    def _():
        acc[...] = jnp.zeros_like(acc)

    acc[...] += partial

    @pl.when(i == N // RB - 1)
    def _():
        out_ref[...] = (jnp.dot(acc[...], w2_ref[...],
                                preferred_element_type=jnp.float32)
                        * (1.0 / N) + b2_ref[...])


_tc_finish = pl.pallas_call(
    _fin_body,
    grid=(N // RB,),
    in_specs=[
        pl.BlockSpec((NC, RB, D), lambda i: (0, i, 0)),
        pl.BlockSpec((RB, 1), lambda i: (i, 0)),
        pl.BlockSpec((RB, 1), lambda i: (i, 0)),
        pl.BlockSpec((RB, 1), lambda i: (i, 0)),
        pl.BlockSpec((RB, 1), lambda i: (i, 0)),
        pl.BlockSpec((1, D), lambda i: (0, 0)),
        pl.BlockSpec((D, D), lambda i: (0, 0)),
        pl.BlockSpec((1, D), lambda i: (0, 0)),
    ],
    out_specs=pl.BlockSpec((1, D), lambda i: (0, 0)),
    out_shape=jax.ShapeDtypeStruct((1, D), jnp.float32),
    scratch_shapes=[pltpu.VMEM((1, D), jnp.float32)],
)


def kernel(features, edge_index, W1, b1, W2, b2):
    pad_vals = jnp.arange(E_PAD - E, dtype=jnp.int32) % (N_PAD - N) + N
    ei = jnp.concatenate(
        [edge_index.astype(jnp.int32),
         jnp.stack([pad_vals, pad_vals])], axis=1)
    dego, degi = _sc_degrees(ei.reshape(2, NS, CH_A, K))
    x1, inv_out, inv_in = _tc_layer1(
        features, dego.reshape(N_PAD, 1), degi.reshape(N_PAD, 1), W1)
    aggp, sp0, sp1 = _sc_edge_pass(
        ei.reshape(2, E_PAD // K, 1, K), x1, inv_in.reshape(N_PAD))
    return _tc_finish(aggp, sp0.reshape(N_PAD, 1), sp1.reshape(N_PAD, 1),
                      inv_out, inv_in, b1.reshape(1, D), W2, b2.reshape(1, D))


# speedup vs baseline: 2.0213x; 1.0190x over previous
import functools

import jax
import jax.numpy as jnp
from jax import lax
from jax.experimental import pallas as pl
from jax.experimental.pallas import tpu as pltpu
from jax.experimental.pallas import tpu_sc as plsc

N = 10000
E = 320000
D = 128
NC = 2
NS = 16
NW = NC * NS
K = 128
N_PAD = 10240
PAD_T = N_PAD // NS
E_PAD = 323584
CH_A = E_PAD // NS // K
CH_C = E_PAD // NW // K

_mesh = plsc.VectorSubcoreMesh(
    core_axis_name="c", subcore_axis_name="s", num_cores=NC, num_subcores=NS)


@functools.partial(
    pl.kernel,
    out_type=[jax.ShapeDtypeStruct((N_PAD,), jnp.float32),
              jax.ShapeDtypeStruct((N_PAD,), jnp.float32)],
    mesh=_mesh,
    scratch_types=[
        pltpu.VMEM_SHARED((N_PAD,), jnp.float32),
        pltpu.VMEM((PAD_T,), jnp.float32),
        pltpu.VMEM((K,), jnp.float32),
        pltpu.VMEM((2, CH_A, K), jnp.int32),
        pltpu.SemaphoreType.DMA,
    ],
)
def _sc_degrees(ei, deg_out, deg_in, hist, zb, ones, idxs, sem):
    cid = lax.axis_index("c")
    sid = lax.axis_index("s")

    def _z(i, _):
        zb[pl.ds(i * 16, 16)] = jnp.zeros((16,), jnp.float32)
        return 0
    lax.fori_loop(0, PAD_T // 16, _z, 0)

    def _o(i, _):
        ones[pl.ds(i * 16, 16)] = jnp.ones((16,), jnp.float32)
        return 0
    lax.fori_loop(0, K // 16, _o, 0)

    pltpu.sync_copy(zb, hist.at[pl.ds(sid * PAD_T, PAD_T)])
    pltpu.sync_copy(ei.at[:, sid], idxs)
    plsc.subcore_barrier()

    def _body(c, _):
        pltpu.async_copy(ones, hist.at[idxs.at[cid, c]], sem, add=True)
        return 0
    lax.fori_loop(0, CH_A, _body, 0)

    def _drain(c, _):
        pltpu.make_async_copy(ones, hist.at[idxs.at[cid, 0]], sem).wait()
        return 0
    lax.fori_loop(0, CH_A, _drain, 0)

    plsc.subcore_barrier()

    @pl.when(cid == 0)
    def _():
        pltpu.sync_copy(hist.at[pl.ds(sid * PAD_T, PAD_T)],
                        deg_out.at[pl.ds(sid * PAD_T, PAD_T)])

    @pl.when(cid == 1)
    def _():
        pltpu.sync_copy(hist.at[pl.ds(sid * PAD_T, PAD_T)],
                        deg_in.at[pl.ds(sid * PAD_T, PAD_T)])


RB = 2000
RP = N_PAD // 5


def _l1_body(f_ref, dop_ref, dip_ref, w1_ref, x1_ref, invo_ref, invi_ref):
    io = lax.rsqrt(jnp.maximum(dop_ref[...], 1.0))
    x1_ref[...] = jnp.dot(f_ref[...] * io, w1_ref[...],
                          preferred_element_type=jnp.float32)
    invo_ref[...] = io
    invi_ref[...] = lax.rsqrt(jnp.maximum(dip_ref[...], 1.0))


_tc_layer1 = pl.pallas_call(
    _l1_body,
    grid=(N_PAD // RP,),
    in_specs=[
        pl.BlockSpec((RP, D), lambda i: (i, 0)),
        pl.BlockSpec((RP, 1), lambda i: (i, 0)),
        pl.BlockSpec((RP, 1), lambda i: (i, 0)),
        pl.BlockSpec((D, D), lambda i: (0, 0)),
    ],
    out_specs=[
        pl.BlockSpec((RP, D), lambda i: (i, 0)),
        pl.BlockSpec((RP, 1), lambda i: (i, 0)),
        pl.BlockSpec((RP, 1), lambda i: (i, 0)),
    ],
    out_shape=[
        jax.ShapeDtypeStruct((N_PAD, D), jnp.float32),
        jax.ShapeDtypeStruct((N_PAD, 1), jnp.float32),
        jax.ShapeDtypeStruct((N_PAD, 1), jnp.float32),
    ],
)


@functools.partial(
    pl.kernel,
    out_type=[jax.ShapeDtypeStruct((NC, N_PAD, D), jnp.float32),
              jax.ShapeDtypeStruct((N_PAD,), jnp.float32),
              jax.ShapeDtypeStruct((N_PAD,), jnp.float32)],
    mesh=_mesh,
    scratch_types=[
        pltpu.VMEM_SHARED((N_PAD, D), jnp.float32),
        pltpu.VMEM_SHARED((N_PAD,), jnp.float32),
        pltpu.VMEM_SHARED((N_PAD,), jnp.float32),
        pltpu.VMEM((32, D), jnp.float32),
        pltpu.VMEM((PAD_T,), jnp.float32),
        pltpu.VMEM((2, 1, K), jnp.int32),
        pltpu.VMEM((2, 1, K), jnp.int32),
        pltpu.VMEM((2, 1, K), jnp.int32),
        pltpu.VMEM((K, D), jnp.float32),
        pltpu.VMEM((K, D), jnp.float32),
        pltpu.VMEM((K,), jnp.float32),
        pltpu.VMEM((K,), jnp.float32),
        pltpu.SemaphoreType.DMA,
        pltpu.SemaphoreType.DMA,
        pltpu.SemaphoreType.DMA,
        pltpu.SemaphoreType.DMA,
        pltpu.SemaphoreType.DMA,
        pltpu.SemaphoreType.DMA,
        pltpu.SemaphoreType.DMA,
        pltpu.SemaphoreType.DMA,
        pltpu.SemaphoreType.DMA,
        pltpu.SemaphoreType.DMA,
        pltpu.SemaphoreType.DMA,
    ],
)
def _sc_edge_pass(ei, x1, invin, aggp, sp0, sp1,
                  agg_sh, s_sh, inv_sh, zb2, zb1, i0, i1, i2,
                  r0, r1, v0, v1, si0, si1, si2, sr0, sr1, sv0, sv1,
                  ssr0, ssr1, ssv0, ssv1):
    cid = lax.axis_index("c")
    sid = lax.axis_index("s")

    def _zr(r, _):
        def _zc(c, _):
            zb2[r, pl.ds(c * 16, 16)] = jnp.zeros((16,), jnp.float32)
            return 0
        lax.fori_loop(0, D // 16, _zc, 0)
        return 0
    lax.fori_loop(0, 32, _zr, 0)

    def _z1(i, _):
        zb1[pl.ds(i * 16, 16)] = jnp.zeros((16,), jnp.float32)
        return 0
    lax.fori_loop(0, PAD_T // 16, _z1, 0)

    for j in range(PAD_T // 32):
        pltpu.sync_copy(zb2, agg_sh.at[pl.ds(sid * PAD_T + j * 32, 32), :])
    pltpu.sync_copy(zb1, s_sh.at[pl.ds(sid * PAD_T, PAD_T)])
    pltpu.sync_copy(invin.at[pl.ds(sid * PAD_T, PAD_T)],
                    inv_sh.at[pl.ds(sid * PAD_T, PAD_T)])

    idxs = (i0, i1, i2)
    rows = (r0, r1)
    vals = (v0, v1)
    isem = (si0, si1, si2)
    rsem = (sr0, sr1)
    vsem = (sv0, sv1)
    srsem = (ssr0, ssr1)
    svsem = (ssv0, ssv1)
    crow0 = (cid * NS + sid) * CH_C

    def load_idx(c, j):
        pltpu.async_copy(ei.at[:, crow0 + c], idxs[j], isem[j])

    def wait_idx(j):
        pltpu.make_async_copy(ei.at[:, 0], idxs[j], isem[j]).wait()

    def gath(j, b):
        pltpu.async_copy(x1.at[idxs[j].at[0, 0]], rows[b], rsem[b])
        pltpu.async_copy(inv_sh.at[idxs[j].at[1, 0]], vals[b], vsem[b])

    def drain_gath(j, b):
        pltpu.make_async_copy(x1.at[idxs[j].at[0, 0]], rows[b], rsem[b]).wait()
        pltpu.make_async_copy(inv_sh.at[idxs[j].at[1, 0]], vals[b],
                              vsem[b]).wait()

    def scat(j, b):
        pltpu.async_copy(rows[b], agg_sh.at[idxs[j].at[1, 0]], srsem[b],
                         add=True)
        pltpu.async_copy(vals[b], s_sh.at[idxs[j].at[0, 0]], svsem[b],
                         add=True)

    def drain_scat(b):
        pltpu.make_async_copy(rows[b], agg_sh.at[i0.at[1, 0]],
                              srsem[b]).wait()
        pltpu.make_async_copy(vals[b], s_sh.at[i0.at[0, 0]],
                              svsem[b]).wait()

    load_idx(0, 0)
    load_idx(1, 1)
    wait_idx(0)
    plsc.subcore_barrier()
    gath(0, 0)

    def stage(c, j, b, first=False, load=True, gather=True):
        if not first:
            drain_scat(b ^ 1)
        if load:
            load_idx(c + 2, (j + 2) % 3)
        if gather:
            wait_idx((j + 1) % 3)
            gath((j + 1) % 3, b ^ 1)
        drain_gath(j, b)
        scat(j, b)

    stage(0, 0, 0, first=True)

    def _body(m, _):
        c = 6 * m + 1
        for k in range(6):
            stage(c + k, (1 + k) % 3, (1 + k) % 2)
        return 0
    lax.fori_loop(0, (CH_C - 7) // 6, _body, 0)

    stage(CH_C - 6, 1, 1)
    stage(CH_C - 5, 2, 0)
    stage(CH_C - 4, 0, 1)
    stage(CH_C - 3, 1, 0)
    stage(CH_C - 2, 2, 1, load=False)
    stage(CH_C - 1, 0, 0, load=False, gather=False)
    drain_scat(0)

    plsc.subcore_barrier()
    pltpu.sync_copy(agg_sh.at[pl.ds(sid * PAD_T, PAD_T), :],
                    aggp.at[cid, pl.ds(sid * PAD_T, PAD_T), :])

    @pl.when(cid == 0)
    def _():
        pltpu.sync_copy(s_sh.at[pl.ds(sid * PAD_T, PAD_T)],
                        sp0.at[pl.ds(sid * PAD_T, PAD_T)])

    @pl.when(cid == 1)
    def _():
        pltpu.sync_copy(s_sh.at[pl.ds(sid * PAD_T, PAD_T)],
                        sp1.at[pl.ds(sid * PAD_T, PAD_T)])


def _fin_body(aggp_ref, sp0_ref, sp1_ref, invo_ref, invi_ref, b1_ref, w2_ref,
              b2_ref, out_ref, acc):
    i = pl.program_id(0)
    agg = aggp_ref[0] + aggp_ref[1]
    h = jnp.maximum(agg * invi_ref[...] + b1_ref[...], 0.0)
    w = invo_ref[...] * (sp0_ref[...] + sp1_ref[...])
    partial = jnp.sum(w * h, axis=0, keepdims=True)

    @pl.when(i == 0)
    def _():
        acc[...] = jnp.zeros_like(acc)

    acc[...] += partial

    @pl.when(i == N // RB - 1)
    def _():
        out_ref[...] = (jnp.dot(acc[...], w2_ref[...],
                                preferred_element_type=jnp.float32)
                        * (1.0 / N) + b2_ref[...])


_tc_finish = pl.pallas_call(
    _fin_body,
    grid=(N // RB,),
    in_specs=[
        pl.BlockSpec((NC, RB, D), lambda i: (0, i, 0)),
        pl.BlockSpec((RB, 1), lambda i: (i, 0)),
        pl.BlockSpec((RB, 1), lambda i: (i, 0)),
        pl.BlockSpec((RB, 1), lambda i: (i, 0)),
        pl.BlockSpec((RB, 1), lambda i: (i, 0)),
        pl.BlockSpec((1, D), lambda i: (0, 0)),
        pl.BlockSpec((D, D), lambda i: (0, 0)),
        pl.BlockSpec((1, D), lambda i: (0, 0)),
    ],
    out_specs=pl.BlockSpec((1, D), lambda i: (0, 0)),
    out_shape=jax.ShapeDtypeStruct((1, D), jnp.float32),
    scratch_shapes=[pltpu.VMEM((1, D), jnp.float32)],
)


def kernel(features, edge_index, W1, b1, W2, b2):
    pad_vals = jnp.arange(E_PAD - E, dtype=jnp.int32) % (N_PAD - N) + N
    ei = jnp.concatenate(
        [edge_index.astype(jnp.int32),
         jnp.stack([pad_vals, pad_vals])], axis=1)
    dego, degi = _sc_degrees(ei.reshape(2, NS, CH_A, K))
    x1, inv_out, inv_in = _tc_layer1(
        features, dego.reshape(N_PAD, 1), degi.reshape(N_PAD, 1), W1)
    aggp, sp0, sp1 = _sc_edge_pass(
        ei.reshape(2, E_PAD // K, 1, K), x1, inv_in.reshape(N_PAD))
    return _tc_finish(aggp, sp0.reshape(N_PAD, 1), sp1.reshape(N_PAD, 1),
                      inv_out, inv_in, b1.reshape(1, D), W2, b2.reshape(1, D))
